# baseline (speedup 1.0000x reference)
import functools
import math

import jax
import jax.numpy as jnp
from jax import lax
from jax.experimental import pallas as pl
from jax.experimental.pallas import tpu as pltpu
from jax.experimental.pallas import tpu_sc as plsc

D_MODEL = 64
SCALE = math.sqrt(D_MODEL)

NC = 2
NS = 16
NW = NC * NS

CHUNK = 128
NBUF = 4


@functools.lru_cache(maxsize=None)
def _build(B, D):
    assert B % (NW * CHUNK) == 0
    cpw = B // (NW * CHUNK)
    rows_pw = cpw * CHUNK
    n_groups = cpw // NBUF
    assert cpw % NBUF == 0 and n_groups >= 2

    mesh = plsc.VectorSubcoreMesh(core_axis_name="c", subcore_axis_name="s")

    @functools.partial(
        pl.kernel,
        mesh=mesh,
        out_type=jax.ShapeDtypeStruct((B, D), jnp.float32),
        compiler_params=pltpu.CompilerParams(use_tc_tiling_on_sc=False),
        scratch_types=(
            [pltpu.VMEM((cpw, CHUNK), jnp.int32)]
            + [pltpu.VMEM((CHUNK, D), jnp.float32) for _ in range(2 * NBUF)]
            + [pltpu.SemaphoreType.DMA for _ in range(2 * NBUF)]
        ),
    )
    def embed(lut_hbm, idx_hbm, out_hbm,
              idx_v,
              g0, g1, g2, g3, o0, o1, o2, o3,
              gs0, gs1, gs2, gs3, os0, os1, os2, os3):
        gbuf = (g0, g1, g2, g3)
        obuf = (o0, o1, o2, o3)
        gsem = (gs0, gs1, gs2, gs3)
        osem = (os0, os1, os2, os3)

        wid = lax.axis_index("s") * NC + lax.axis_index("c")
        row0 = wid * rows_pw

        pltpu.sync_copy(idx_hbm.at[wid], idx_v)

        def start_gather(c, b):
            pltpu.async_copy(lut_hbm.at[idx_v.at[c]], gbuf[b], gsem[b])

        def wait_gather(b):
            pltpu.make_async_copy(lut_hbm.at[idx_v.at[0]], gbuf[b], gsem[b]).wait()

        def start_out(c, b):
            pltpu.async_copy(obuf[b], out_hbm.at[pl.ds(row0 + c * CHUNK, CHUNK)],
                             osem[b])

        def wait_out(b):
            pltpu.make_async_copy(obuf[b], out_hbm.at[pl.ds(0, CHUNK)],
                                  osem[b]).wait()

        def scale_chunk(b):
            src = gbuf[b]
            dst = obuf[b]

            def body(i, carry):
                r = i * 4
                for rr in range(4):
                    for k in range(D // 16):
                        sl = pl.ds(k * 16, 16)
                        dst[r + rr, sl] = src[r + rr, sl] * SCALE
                return carry

            lax.fori_loop(0, CHUNK // 4, body, 0)

        for b in range(NBUF):
            start_gather(b, b)

        for b in range(NBUF):
            wait_gather(b)
            scale_chunk(b)
            start_out(b, b)
            start_gather(b + NBUF, b)

        def group(g, carry):
            for b in range(NBUF):
                c = g * NBUF + b
                wait_gather(b)
                wait_out(b)
                scale_chunk(b)
                start_out(c, b)
                start_gather(c + NBUF, b)
            return carry

        lax.fori_loop(1, n_groups - 1, group, 0)

        for b in range(NBUF):
            c = (n_groups - 1) * NBUF + b
            wait_gather(b)
            wait_out(b)
            scale_chunk(b)
            start_out(c, b)

        for b in range(NBUF):
            wait_out(b)

    return embed


def kernel(input, lut):
    lead_shape = input.shape
    idx = input.reshape(-1).astype(jnp.int32)
    B = idx.shape[0]
    D = lut.shape[1]
    cpw = B // (NW * CHUNK)
    idx3 = idx.reshape(NW, cpw, CHUNK)
    out = _build(B, D)(lut, idx3)
    return out.reshape(*lead_shape, D)
